# baseline (speedup 1.0000x reference)
import functools

import jax
import jax.numpy as jnp
from jax import lax
from jax.experimental import pallas as pl
from jax.experimental.pallas import tpu as pltpu
from jax.experimental.pallas import tpu_sc as plsc

NQ = 4
K = 8192
D = 256
N = 8192

KBLK = 512
RBLK = 512

NC = 2
NS = 16
NW = NC * NS
TPW = N // NW
CHUNK = 128
NCHUNK = TPW // CHUNK
BINS_PER_SUB = K // NS


def _argmin_body(x_ref, cb_ref, rn_ref, cn_ref, idx_ref,
                 best_val, best_idx):
    kb = pl.program_id(0)
    rb = pl.program_id(1)
    nk = pl.num_programs(0)

    rs = pl.ds(rb * RBLK, RBLK)
    x = x_ref[rs, :]
    cbt = cb_ref[...]
    dot = lax.dot_general(x, cbt, (((1,), (0,)), ((), ())),
                          preferred_element_type=jnp.float32)
    scores = (rn_ref[rs, :] - 2.0 * dot) + cn_ref[:, :]

    lmin = jnp.min(scores, axis=1, keepdims=True)
    it = lax.broadcasted_iota(jnp.int32, (RBLK, KBLK), 1)
    larg = jnp.min(jnp.where(scores == lmin, it, K), axis=1,
                   keepdims=True) + kb * KBLK

    @pl.when(kb == 0)
    def _():
        best_val[rs, :] = lmin
        best_idx[rs, :] = larg

    @pl.when(kb > 0)
    def _():
        prev_v = best_val[rs, :]
        prev_i = best_idx[rs, :]
        better = lmin < prev_v
        best_val[rs, :] = jnp.where(better, lmin, prev_v)
        best_idx[rs, :] = jnp.where(better, larg, prev_i)

    @pl.when(kb == nk - 1)
    def _():
        idx_ref[rs, :] = best_idx[rs, :]


def _argmin_stage(flat_x_bf, codebook_t_bf, rownorm, colnorm):
    grid = (K // KBLK, N // RBLK)
    return pl.pallas_call(
        _argmin_body,
        grid=grid,
        in_specs=[
            pl.BlockSpec((N, D), lambda kb, rb: (0, 0)),
            pl.BlockSpec((D, KBLK), lambda kb, rb: (0, kb)),
            pl.BlockSpec((N, 1), lambda kb, rb: (0, 0)),
            pl.BlockSpec((1, KBLK), lambda kb, rb: (0, kb)),
        ],
        out_specs=pl.BlockSpec((N, 1), lambda kb, rb: (0, 0)),
        out_shape=jax.ShapeDtypeStruct((N, 1), jnp.int32),
        scratch_shapes=[
            pltpu.VMEM((N, 1), jnp.float32),
            pltpu.VMEM((N, 1), jnp.int32),
        ],
        compiler_params=pltpu.CompilerParams(
            dimension_semantics=("arbitrary", "arbitrary")),
    )(flat_x_bf, codebook_t_bf, rownorm, colnorm)


def _sc_body(idx_hbm, cb_hbm, res_hbm, zeros_hbm,
             out_res, out_flags,
             idx_v, rows_v, res_v, bins_v, sem):
    cid = lax.axis_index("c")
    sid = lax.axis_index("s")
    wid = cid * NS + sid

    pltpu.sync_copy(zeros_hbm, bins_v)
    pltpu.sync_copy(idx_hbm.at[pl.ds(wid * NCHUNK, NCHUNK)], idx_v)
    ones16 = jnp.ones((16,), jnp.float32)

    for c in range(NCHUNK):
        tok0 = wid * TPW + c * CHUNK
        gcopy = pltpu.async_copy(cb_hbm.at[idx_v.at[c]], rows_v, sem)
        pltpu.sync_copy(res_hbm.at[pl.ds(tok0, CHUNK)], res_v)
        gcopy.wait()

        def body(r, _):
            for d in range(D // 16):
                sl = pl.ds(d * 16, 16)
                x = res_v[r, sl]
                q = rows_v[r, sl]
                res_v[r, sl] = x - (x + (q - x))
            return 0

        lax.fori_loop(0, CHUNK, body, 0)
        pltpu.sync_copy(res_v, out_res.at[pl.ds(tok0, CHUNK)])
        for j in range(CHUNK // 16):
            plsc.store_scatter(bins_v, [idx_v[c, pl.ds(j * 16, 16)]], ones16)

    pltpu.sync_copy(bins_v, out_flags.at[wid])


def _sc_stage(idx_2d, codebook, flat_res, zeros_in):
    mesh = plsc.VectorSubcoreMesh(core_axis_name="c", subcore_axis_name="s")
    fn = pl.kernel(
        _sc_body,
        out_type=[
            jax.ShapeDtypeStruct((N, D), jnp.float32),
            jax.ShapeDtypeStruct((NW, K), jnp.float32),
        ],
        mesh=mesh,
        scratch_types=[
            pltpu.VMEM((NCHUNK, CHUNK), jnp.int32),
            pltpu.VMEM((CHUNK, D), jnp.float32),
            pltpu.VMEM((CHUNK, D), jnp.float32),
            pltpu.VMEM((K,), jnp.float32),
            pltpu.SemaphoreType.DMA,
        ],
        compiler_params=pltpu.CompilerParams(needs_layout_passes=False),
    )
    return fn(idx_2d, codebook, flat_res, zeros_in)


def _epilogue_body(x0_ref, r4_ref, flags_ref, rn_ref,
                   qsum_ref, usage_ref, loss_ref):
    x0 = x0_ref[...]
    r4 = r4_ref[...]
    qsum_ref[...] = x0 - r4
    ssq4 = jnp.sum(r4 * r4)
    total = jnp.sum(rn_ref[...]) + ssq4
    loss_ref[0, 0] = 0.25 * total / (N * D)
    for i in range(NQ):
        m = jnp.max(flags_ref[i, :, :], axis=0)
        usage_ref[i, 0] = jnp.sum((m > 0.0).astype(jnp.float32)) / K


def _epilogue(flat_x0, flat_r4, flags4, rn_cat):
    return pl.pallas_call(
        _epilogue_body,
        in_specs=[
            pl.BlockSpec((N, D), lambda: (0, 0)),
            pl.BlockSpec((N, D), lambda: (0, 0)),
            pl.BlockSpec((NQ, NW, K), lambda: (0, 0, 0)),
            pl.BlockSpec((N, NQ - 1), lambda: (0, 0)),
        ],
        out_specs=[
            pl.BlockSpec((N, D), lambda: (0, 0)),
            pl.BlockSpec(memory_space=pltpu.SMEM),
            pl.BlockSpec(memory_space=pltpu.SMEM),
        ],
        out_shape=[
            jax.ShapeDtypeStruct((N, D), jnp.float32),
            jax.ShapeDtypeStruct((NQ, 1), jnp.float32),
            jax.ShapeDtypeStruct((1, 1), jnp.float32),
        ],
    )(flat_x0, flat_r4, flags4, rn_cat)


def kernel(inputs, codebooks):
    B, T, _ = inputs.shape
    flat = inputs.reshape(N, D)
    zeros_in = jnp.zeros((K,), jnp.float32)

    residual = flat
    cbt_bf = jnp.swapaxes(codebooks, 1, 2).astype(jnp.bfloat16)
    idx_list = []
    flags_list = []
    rn_list = []
    for i in range(NQ):
        rownorm = jnp.sum(residual ** 2, axis=1, keepdims=True)
        colnorm = jnp.sum(codebooks[i] ** 2, axis=1)[None, :]
        rn_list.append(rownorm)
        idx = _argmin_stage(residual.astype(jnp.bfloat16), cbt_bf[i],
                            rownorm, colnorm)
        idx_list.append(idx)
        idx_2d = idx.reshape(N // CHUNK, CHUNK)
        residual, flags = _sc_stage(idx_2d, codebooks[i], residual, zeros_in)
        flags_list.append(flags)

    flags4 = jnp.stack(flags_list, axis=0)
    rn_cat = jnp.concatenate(rn_list[1:], axis=1)

    qsum, usage, loss = _epilogue(flat, residual, flags4, rn_cat)

    quantized_sum = qsum.reshape(B, T, D)
    encodings = jnp.stack([ix.reshape(B, T) for ix in idx_list], axis=0)
    usage_ratios = usage.reshape(NQ)
    total_loss = loss.reshape(())
    return quantized_sum, encodings, usage_ratios, total_loss
